# baseline (speedup 1.0000x reference)
import jax
import jax.numpy as jnp
from jax.experimental import pallas as pl

_T = 2048
_E = 768
_DIM = 1024
_HEADS = 16
_DH = 32
_DIM_HEADS = _HEADS * _DH
_OUP = 1024
_NEG = -5e4
_TBLK = 512


def _proj_kernel(x_ref, wt3_ref, wqk_ref, wv_ref, qk_ref, v_ref):
    h = jax.lax.dot_general(
        x_ref[...], wt3_ref[...], (((1,), (1,)), ((), ())),
        preferred_element_type=jnp.float32)
    qk_ref[...] = jax.lax.dot_general(
        h, wqk_ref[...], (((1,), (1,)), ((), ())),
        preferred_element_type=jnp.float32)
    v_ref[...] = jax.lax.dot_general(
        h, wv_ref[...], (((1,), (1,)), ((), ())),
        preferred_element_type=jnp.float32)


def _attn_kernel(qk_ref, va_ref, wo_ref, bout_ref, out_ref):
    head = pl.program_id(0)
    qk = qk_ref[0]
    va = va_ref[0]
    inv_sqrt = _DH ** -0.5
    norm = jnp.sqrt(jnp.sum(qk * qk, axis=-1, keepdims=True))
    k = qk / jnp.maximum(norm, 1e-12)
    s = jax.lax.dot_general(
        qk.astype(jnp.bfloat16), k.astype(jnp.bfloat16),
        (((1,), (1,)), ((), ())),
        preferred_element_type=jnp.float32) * inv_sqrt
    m = norm * inv_sqrt
    e = jnp.exp(s - m).astype(jnp.bfloat16)
    num = jnp.dot(e, va, preferred_element_type=jnp.float32)
    denom = num[:, _DH:_DH + 1] - 1.0
    o = (num[:, :_DH] - va[:, :_DH].astype(jnp.float32)) / denom
    contrib = jax.lax.dot_general(
        o, wo_ref[0], (((1,), (1,)), ((), ())),
        preferred_element_type=jnp.float32)

    @pl.when(head == 0)
    def _():
        out_ref[...] = bout_ref[...] + contrib

    @pl.when(head != 0)
    def _():
        out_ref[...] += contrib


def kernel(x, W_temp3, W_toqk, W_tov, W_out, b_out):
    x2 = x[0]
    qk, v = pl.pallas_call(
        _proj_kernel,
        grid=(_T // _TBLK,),
        in_specs=[
            pl.BlockSpec((_TBLK, _E), lambda i: (i, 0)),
            pl.BlockSpec((_DIM, _E), lambda i: (0, 0)),
            pl.BlockSpec((_DIM_HEADS, _DIM), lambda i: (0, 0)),
            pl.BlockSpec((_DIM_HEADS, _DIM), lambda i: (0, 0)),
        ],
        out_specs=[
            pl.BlockSpec((_TBLK, _DIM_HEADS), lambda i: (i, 0)),
            pl.BlockSpec((_TBLK, _DIM_HEADS), lambda i: (i, 0)),
        ],
        out_shape=[
            jax.ShapeDtypeStruct((_T, _DIM_HEADS), jnp.float32),
            jax.ShapeDtypeStruct((_T, _DIM_HEADS), jnp.float32),
        ],
    )(x2, W_temp3, W_toqk, W_tov)

    qk3 = qk.reshape(_T, _HEADS, _DH).transpose(1, 0, 2)
    v3 = v.reshape(_T, _HEADS, _DH).transpose(1, 0, 2)
    ones = jnp.ones((_HEADS, _T, 1), jnp.float32)
    zeros = jnp.zeros((_HEADS, _T, _DH - 1), jnp.float32)
    va3 = jnp.concatenate([v3, ones, zeros], axis=-1).astype(jnp.bfloat16)
    wo3 = W_out.reshape(_OUP, _HEADS, _DH).transpose(1, 0, 2)

    out = pl.pallas_call(
        _attn_kernel,
        grid=(_HEADS,),
        in_specs=[
            pl.BlockSpec((1, _T, _DH), lambda h: (h, 0, 0)),
            pl.BlockSpec((1, _T, 2 * _DH), lambda h: (h, 0, 0)),
            pl.BlockSpec((1, _OUP, _DH), lambda h: (h, 0, 0)),
            pl.BlockSpec((1, _OUP), lambda h: (0, 0)),
        ],
        out_specs=pl.BlockSpec((_T, _OUP), lambda h: (0, 0)),
        out_shape=jax.ShapeDtypeStruct((_T, _OUP), jnp.float32),
    )(qk3, va3, wo3, b_out.reshape(1, _OUP))

    return out.reshape(1, _T, _OUP)


# speedup vs baseline: 1.3145x; 1.3145x over previous
import jax
import jax.numpy as jnp
from jax.experimental import pallas as pl

_T = 2048
_E = 768
_DIM = 1024
_HEADS = 16
_DH = 32
_DIM_HEADS = _HEADS * _DH
_OUP = 1024
_TBLK = 512
_LOG2E = 1.4426950408889634


def _proj_kernel(x_ref, wt3_ref, wqk_ref, wv_ref, qk_ref, v_ref):
    h = jax.lax.dot_general(
        x_ref[...], wt3_ref[...], (((1,), (1,)), ((), ())),
        preferred_element_type=jnp.float32)
    qk_ref[...] = jax.lax.dot_general(
        h, wqk_ref[...], (((1,), (1,)), ((), ())),
        preferred_element_type=jnp.float32)
    v_ref[...] = jax.lax.dot_general(
        h, wv_ref[...], (((1,), (1,)), ((), ())),
        preferred_element_type=jnp.float32)


def _attn_kernel(qk_ref, va_ref, o_ref):
    qk = qk_ref[0]
    va = va_ref[0]
    c = _DH ** -0.5 * _LOG2E
    norm = jnp.sqrt(jnp.sum(qk * qk, axis=-1, keepdims=True))
    k = qk / jnp.maximum(norm, 1e-12)
    zpad = jnp.zeros((_T, _DH - 1), jnp.float32)
    lhs = jnp.concatenate([qk * c, -norm * c, zpad], axis=1)
    rhs = jnp.concatenate([k, jnp.ones((_T, 1), jnp.float32), zpad], axis=1)
    s = jax.lax.dot_general(
        lhs.astype(jnp.bfloat16), rhs.astype(jnp.bfloat16),
        (((1,), (1,)), ((), ())),
        preferred_element_type=jnp.float32)
    e = jnp.exp2(s.astype(jnp.bfloat16))
    num = jnp.dot(e, va, preferred_element_type=jnp.float32)
    denom = num[:, _DH:_DH + 1] - 1.0
    o_ref[0] = (num[:, :_DH] - va[:, :_DH].astype(jnp.float32)) / denom


def _outproj_kernel(o_ref, wo_ref, b_ref, out_ref):
    out_ref[...] = jax.lax.dot_general(
        o_ref[...], wo_ref[...], (((1,), (1,)), ((), ())),
        preferred_element_type=jnp.float32) + b_ref[...]


def kernel(x, W_temp3, W_toqk, W_tov, W_out, b_out):
    x2 = x[0]
    qk, v = pl.pallas_call(
        _proj_kernel,
        grid=(_T // _TBLK,),
        in_specs=[
            pl.BlockSpec((_TBLK, _E), lambda i: (i, 0)),
            pl.BlockSpec((_DIM, _E), lambda i: (0, 0)),
            pl.BlockSpec((_DIM_HEADS, _DIM), lambda i: (0, 0)),
            pl.BlockSpec((_DIM_HEADS, _DIM), lambda i: (0, 0)),
        ],
        out_specs=[
            pl.BlockSpec((_TBLK, _DIM_HEADS), lambda i: (i, 0)),
            pl.BlockSpec((_TBLK, _DIM_HEADS), lambda i: (i, 0)),
        ],
        out_shape=[
            jax.ShapeDtypeStruct((_T, _DIM_HEADS), jnp.float32),
            jax.ShapeDtypeStruct((_T, _DIM_HEADS), jnp.float32),
        ],
    )(x2, W_temp3, W_toqk, W_tov)

    qk3 = qk.reshape(_T, _HEADS, _DH).transpose(1, 0, 2)
    v3 = v.reshape(_T, _HEADS, _DH).transpose(1, 0, 2)
    ones = jnp.ones((_HEADS, _T, 1), jnp.float32)
    zeros = jnp.zeros((_HEADS, _T, _DH - 1), jnp.float32)
    va3 = jnp.concatenate([v3, ones, zeros], axis=-1).astype(jnp.bfloat16)

    o3 = pl.pallas_call(
        _attn_kernel,
        grid=(_HEADS,),
        in_specs=[
            pl.BlockSpec((1, _T, _DH), lambda h: (h, 0, 0)),
            pl.BlockSpec((1, _T, 2 * _DH), lambda h: (h, 0, 0)),
        ],
        out_specs=pl.BlockSpec((1, _T, _DH), lambda h: (h, 0, 0)),
        out_shape=jax.ShapeDtypeStruct((_HEADS, _T, _DH), jnp.float32),
    )(qk3, va3)

    o2 = o3.transpose(1, 0, 2).reshape(_T, _DIM_HEADS)
    out = pl.pallas_call(
        _outproj_kernel,
        grid=(_T // _TBLK,),
        in_specs=[
            pl.BlockSpec((_TBLK, _DIM_HEADS), lambda i: (i, 0)),
            pl.BlockSpec((_OUP, _DIM_HEADS), lambda i: (0, 0)),
            pl.BlockSpec((1, _OUP), lambda i: (0, 0)),
        ],
        out_specs=pl.BlockSpec((_TBLK, _OUP), lambda i: (i, 0)),
        out_shape=jax.ShapeDtypeStruct((_T, _OUP), jnp.float32),
    )(o2, W_out, b_out.reshape(1, _OUP))

    return out.reshape(1, _T, _OUP)


# speedup vs baseline: 1.5474x; 1.1772x over previous
import jax
import jax.numpy as jnp
from jax.experimental import pallas as pl

_T = 2048
_E = 768
_DIM = 1024
_HEADS = 16
_DH = 32
_DIM_HEADS = _HEADS * _DH
_OUP = 1024
_TBLK = 512
_LOG2E = 1.4426950408889634
_C = _DH ** -0.5 * _LOG2E


def _proj_kernel(x_ref, wt3_ref, wqk_ref, wv_ref, lhs_ref, rhs_ref, va_ref):
    h = jax.lax.dot_general(
        x_ref[...], wt3_ref[...], (((1,), (1,)), ((), ())),
        preferred_element_type=jnp.float32)
    qk = jax.lax.dot_general(
        h, wqk_ref[...], (((1,), (1,)), ((), ())),
        preferred_element_type=jnp.float32)
    v = jax.lax.dot_general(
        h, wv_ref[...], (((1,), (1,)), ((), ())),
        preferred_element_type=jnp.float32)
    col = jax.lax.broadcasted_iota(jnp.int32, (_DIM_HEADS, _HEADS), 0)
    hid = jax.lax.broadcasted_iota(jnp.int32, (_DIM_HEADS, _HEADS), 1)
    sel = (col // _DH == hid).astype(jnp.float32)
    nsq = jnp.dot(qk * qk, sel, preferred_element_type=jnp.float32)
    norm = jnp.maximum(jnp.sqrt(nsq), 1e-12)
    invb = jnp.dot(1.0 / norm, sel.T, preferred_element_type=jnp.float32)
    k = qk * invb

    def headmajor(z):
        return z.reshape(_TBLK, _HEADS, _DH).transpose(1, 0, 2)

    ones = jnp.ones((_HEADS, _TBLK, 1), jnp.float32)
    zpad = jnp.zeros((_HEADS, _TBLK, _DH - 1), jnp.float32)
    mcol = (-_C * norm).T[:, :, None]
    lhs_ref[...] = jnp.concatenate(
        [headmajor(_C * qk), mcol, zpad], axis=2).astype(jnp.bfloat16)
    rhs_ref[...] = jnp.concatenate(
        [headmajor(k), ones, zpad], axis=2).astype(jnp.bfloat16)
    va_ref[...] = jnp.concatenate(
        [headmajor(v), ones, zpad], axis=2).astype(jnp.bfloat16)


def _attn_kernel(lhs_ref, rhs_ref, va_ref, o_ref):
    lhs = lhs_ref[0]
    rhs = rhs_ref[0]
    va = va_ref[0]
    s = jax.lax.dot_general(
        lhs, rhs, (((1,), (1,)), ((), ())),
        preferred_element_type=jnp.float32)
    e = jnp.exp2(s.astype(jnp.bfloat16))
    num = jnp.dot(e, va, preferred_element_type=jnp.float32)
    denom = num[:, _DH:_DH + 1] - 1.0
    o_ref[0] = (num[:, :_DH] - va[:, :_DH].astype(jnp.float32)) / denom


def _outproj_kernel(o_ref, wo_ref, b_ref, out_ref):
    o2 = o_ref[...].transpose(1, 0, 2).reshape(_T, _DIM_HEADS)
    out_ref[...] = jax.lax.dot_general(
        o2, wo_ref[...], (((1,), (1,)), ((), ())),
        preferred_element_type=jnp.float32) + b_ref[...]


def kernel(x, W_temp3, W_toqk, W_tov, W_out, b_out):
    x2 = x[0]
    lhs3, rhs3, va3 = pl.pallas_call(
        _proj_kernel,
        grid=(_T // _TBLK,),
        in_specs=[
            pl.BlockSpec((_TBLK, _E), lambda i: (i, 0)),
            pl.BlockSpec((_DIM, _E), lambda i: (0, 0)),
            pl.BlockSpec((_DIM_HEADS, _DIM), lambda i: (0, 0)),
            pl.BlockSpec((_DIM_HEADS, _DIM), lambda i: (0, 0)),
        ],
        out_specs=[
            pl.BlockSpec((_HEADS, _TBLK, 2 * _DH), lambda i: (0, i, 0)),
            pl.BlockSpec((_HEADS, _TBLK, 2 * _DH), lambda i: (0, i, 0)),
            pl.BlockSpec((_HEADS, _TBLK, 2 * _DH), lambda i: (0, i, 0)),
        ],
        out_shape=[
            jax.ShapeDtypeStruct((_HEADS, _T, 2 * _DH), jnp.bfloat16),
            jax.ShapeDtypeStruct((_HEADS, _T, 2 * _DH), jnp.bfloat16),
            jax.ShapeDtypeStruct((_HEADS, _T, 2 * _DH), jnp.bfloat16),
        ],
    )(x2, W_temp3, W_toqk, W_tov)

    o3 = pl.pallas_call(
        _attn_kernel,
        grid=(_HEADS,),
        in_specs=[
            pl.BlockSpec((1, _T, 2 * _DH), lambda h: (h, 0, 0)),
            pl.BlockSpec((1, _T, 2 * _DH), lambda h: (h, 0, 0)),
            pl.BlockSpec((1, _T, 2 * _DH), lambda h: (h, 0, 0)),
        ],
        out_specs=pl.BlockSpec((1, _T, _DH), lambda h: (h, 0, 0)),
        out_shape=jax.ShapeDtypeStruct((_HEADS, _T, _DH), jnp.float32),
    )(lhs3, rhs3, va3)

    out = pl.pallas_call(
        _outproj_kernel,
        in_specs=[
            pl.BlockSpec((_HEADS, _T, _DH), lambda: (0, 0, 0)),
            pl.BlockSpec((_OUP, _DIM_HEADS), lambda: (0, 0)),
            pl.BlockSpec((1, _OUP), lambda: (0, 0)),
        ],
        out_specs=pl.BlockSpec((_T, _OUP), lambda: (0, 0)),
        out_shape=jax.ShapeDtypeStruct((_T, _OUP), jnp.float32),
    )(o3, W_out, b_out.reshape(1, _OUP))

    return out.reshape(1, _T, _OUP)


# speedup vs baseline: 2.0266x; 1.3097x over previous
import jax
import jax.numpy as jnp
from jax.experimental import pallas as pl

_T = 2048
_E = 768
_DIM = 1024
_HEADS = 16
_DH = 32
_DIM_HEADS = _HEADS * _DH
_OUP = 1024
_TBLK = 512
_LOG2E = 1.4426950408889634
_C = _DH ** -0.5 * _LOG2E
_AUG = 2 * _DH


def _proj_kernel(x_ref, wt3_ref, wqk_ref, wv_ref, lhs_ref, rhs_ref, va_ref):
    hT = jax.lax.dot_general(
        wt3_ref[...], x_ref[...], (((1,), (1,)), ((), ())),
        preferred_element_type=jnp.float32)
    qkT = jax.lax.dot_general(
        wqk_ref[...], hT, (((1,), (0,)), ((), ())),
        preferred_element_type=jnp.float32)
    vT = jax.lax.dot_general(
        wv_ref[...], hT, (((1,), (0,)), ((), ())),
        preferred_element_type=jnp.float32)
    hid = jax.lax.broadcasted_iota(jnp.int32, (_HEADS, _DIM_HEADS), 0)
    row = jax.lax.broadcasted_iota(jnp.int32, (_HEADS, _DIM_HEADS), 1)
    selT = (row // _DH == hid).astype(jnp.float32)
    nsqT = jax.lax.dot_general(
        selT, qkT * qkT, (((1,), (0,)), ((), ())),
        preferred_element_type=jnp.float32)
    normT = jnp.maximum(jnp.sqrt(nsqT), 1e-12)
    invbT = jax.lax.dot_general(
        selT, 1.0 / normT, (((0,), (0,)), ((), ())),
        preferred_element_type=jnp.float32)
    kT = qkT * invbT

    def split(z):
        return z.reshape(_HEADS, _DH, _TBLK)

    e0 = (jax.lax.broadcasted_iota(jnp.int32, (_HEADS, 8, _TBLK), 1) == 0)
    e0 = e0.astype(jnp.float32)
    mrow = jnp.broadcast_to((-_C * normT)[:, None, :], (_HEADS, 8, _TBLK))
    ztail = jnp.zeros((_HEADS, _AUG - _DH - 8, _TBLK), jnp.float32)
    lhs_ref[...] = jnp.concatenate(
        [split(_C * qkT), mrow * e0, ztail], axis=1).astype(jnp.bfloat16)
    rhs_ref[...] = jnp.concatenate(
        [split(kT), e0, ztail], axis=1).astype(jnp.bfloat16)
    va_ref[...] = jnp.concatenate(
        [split(vT), e0, ztail], axis=1).astype(jnp.bfloat16)


def _attn_kernel(lhs_ref, rhs_ref, va_ref, o_ref):
    lhs = lhs_ref[0]
    rhs = rhs_ref[0]
    va = va_ref[0]
    sT = jax.lax.dot_general(
        rhs, lhs, (((0,), (0,)), ((), ())),
        preferred_element_type=jnp.float32)
    eT = jnp.exp2(sT.astype(jnp.bfloat16))
    numT = jax.lax.dot_general(
        va, eT, (((1,), (0,)), ((), ())),
        preferred_element_type=jnp.float32)
    denom = numT[_DH:_DH + 1, :] - 1.0
    o_ref[0] = (numT[:_DH, :] - va[:_DH, :].astype(jnp.float32)) / denom


def _outproj_kernel(o_ref, wo_ref, b_ref, out_ref):
    oT = o_ref[...].reshape(_DIM_HEADS, _TBLK)
    out_ref[...] = jax.lax.dot_general(
        oT, wo_ref[...], (((0,), (1,)), ((), ())),
        preferred_element_type=jnp.float32) + b_ref[...]


def kernel(x, W_temp3, W_toqk, W_tov, W_out, b_out):
    x2 = x[0]
    lhs3, rhs3, va3 = pl.pallas_call(
        _proj_kernel,
        grid=(_T // _TBLK,),
        in_specs=[
            pl.BlockSpec((_TBLK, _E), lambda i: (i, 0)),
            pl.BlockSpec((_DIM, _E), lambda i: (0, 0)),
            pl.BlockSpec((_DIM_HEADS, _DIM), lambda i: (0, 0)),
            pl.BlockSpec((_DIM_HEADS, _DIM), lambda i: (0, 0)),
        ],
        out_specs=[
            pl.BlockSpec((_HEADS, _AUG, _TBLK), lambda i: (0, 0, i)),
            pl.BlockSpec((_HEADS, _AUG, _TBLK), lambda i: (0, 0, i)),
            pl.BlockSpec((_HEADS, _AUG, _TBLK), lambda i: (0, 0, i)),
        ],
        out_shape=[
            jax.ShapeDtypeStruct((_HEADS, _AUG, _T), jnp.bfloat16),
            jax.ShapeDtypeStruct((_HEADS, _AUG, _T), jnp.bfloat16),
            jax.ShapeDtypeStruct((_HEADS, _AUG, _T), jnp.bfloat16),
        ],
    )(x2, W_temp3, W_toqk, W_tov)

    o3 = pl.pallas_call(
        _attn_kernel,
        grid=(_HEADS,),
        in_specs=[
            pl.BlockSpec((1, _AUG, _T), lambda h: (h, 0, 0)),
            pl.BlockSpec((1, _AUG, _T), lambda h: (h, 0, 0)),
            pl.BlockSpec((1, _AUG, _T), lambda h: (h, 0, 0)),
        ],
        out_specs=pl.BlockSpec((1, _DH, _T), lambda h: (h, 0, 0)),
        out_shape=jax.ShapeDtypeStruct((_HEADS, _DH, _T), jnp.float32),
    )(lhs3, rhs3, va3)

    out = pl.pallas_call(
        _outproj_kernel,
        grid=(_T // _TBLK,),
        in_specs=[
            pl.BlockSpec((_HEADS, _DH, _TBLK), lambda i: (0, 0, i)),
            pl.BlockSpec((_OUP, _DIM_HEADS), lambda i: (0, 0)),
            pl.BlockSpec((1, _OUP), lambda i: (0, 0)),
        ],
        out_specs=pl.BlockSpec((_TBLK, _OUP), lambda i: (i, 0)),
        out_shape=jax.ShapeDtypeStruct((_T, _OUP), jnp.float32),
    )(o3, W_out, b_out.reshape(1, _OUP))

    return out.reshape(1, _T, _OUP)


# speedup vs baseline: 2.1333x; 1.0526x over previous
import jax
import jax.numpy as jnp
from jax.experimental import pallas as pl
from jax.experimental.pallas import tpu as pltpu

_T = 2048
_E = 768
_DIM = 1024
_HEADS = 16
_DH = 32
_DIM_HEADS = _HEADS * _DH
_OUP = 1024
_TBLK = 512
_NBLK = _T // _TBLK
_LOG2E = 1.4426950408889634
_C = _DH ** -0.5 * _LOG2E
_AUG = 2 * _DH


def _mono_kernel(x_ref, wt3_ref, wqk_ref, wv_ref, wo_ref, b_ref, out_ref,
                 lhs_s, rhs_s, va_s, o_s):
    step = pl.program_id(0)

    def proj(i):
        hT = jax.lax.dot_general(
            wt3_ref[...], x_ref[...], (((1,), (1,)), ((), ())),
            preferred_element_type=jnp.float32)
        qkT = jax.lax.dot_general(
            wqk_ref[...], hT, (((1,), (0,)), ((), ())),
            preferred_element_type=jnp.float32)
        vT = jax.lax.dot_general(
            wv_ref[...], hT, (((1,), (0,)), ((), ())),
            preferred_element_type=jnp.float32)
        hid = jax.lax.broadcasted_iota(jnp.int32, (_HEADS, _DIM_HEADS), 0)
        row = jax.lax.broadcasted_iota(jnp.int32, (_HEADS, _DIM_HEADS), 1)
        selT = (row // _DH == hid).astype(jnp.float32)
        nsqT = jax.lax.dot_general(
            selT, qkT * qkT, (((1,), (0,)), ((), ())),
            preferred_element_type=jnp.float32)
        normT = jnp.maximum(jnp.sqrt(nsqT), 1e-12)
        invbT = jax.lax.dot_general(
            selT, 1.0 / normT, (((0,), (0,)), ((), ())),
            preferred_element_type=jnp.float32)
        kT = qkT * invbT

        def split(z):
            return z.reshape(_HEADS, _DH, _TBLK)

        e0 = (jax.lax.broadcasted_iota(jnp.int32, (_HEADS, 8, _TBLK), 1) == 0)
        e0 = e0.astype(jnp.float32)
        mrow = jnp.broadcast_to((-_C * normT)[:, None, :], (_HEADS, 8, _TBLK))
        ztail = jnp.zeros((_HEADS, _AUG - _DH - 8, _TBLK), jnp.float32)
        sl = slice(i * _TBLK, (i + 1) * _TBLK)
        lhs_s[:, :, sl] = jnp.concatenate(
            [split(_C * qkT), mrow * e0, ztail], axis=1).astype(jnp.bfloat16)
        rhs_s[:, :, sl] = jnp.concatenate(
            [split(kT), e0, ztail], axis=1).astype(jnp.bfloat16)
        va_s[:, :, sl] = jnp.concatenate(
            [split(vT), e0, ztail], axis=1).astype(jnp.bfloat16)

    for i in range(_NBLK):
        @pl.when(step == i)
        def _(i=i):
            proj(i)

    @pl.when((step >= _NBLK) & (step < _NBLK + _HEADS))
    def _attn():
        h = step - _NBLK
        lhs = lhs_s[h]
        rhs = rhs_s[h]
        va = va_s[h]
        sT = jax.lax.dot_general(
            rhs, lhs, (((0,), (0,)), ((), ())),
            preferred_element_type=jnp.float32)
        eT = jnp.exp2(sT.astype(jnp.bfloat16))
        numT = jax.lax.dot_general(
            va, eT, (((1,), (0,)), ((), ())),
            preferred_element_type=jnp.float32)
        denom = numT[_DH:_DH + 1, :] - 1.0
        o_s[h] = (numT[:_DH, :] - va[:_DH, :].astype(jnp.float32)) / denom

    for i in range(_NBLK):
        @pl.when(step == _NBLK + _HEADS + i)
        def _(i=i):
            oT = o_s[:, :, i * _TBLK:(i + 1) * _TBLK].reshape(
                _DIM_HEADS, _TBLK)
            out_ref[...] = jax.lax.dot_general(
                oT, wo_ref[...], (((0,), (1,)), ((), ())),
                preferred_element_type=jnp.float32) + b_ref[...]


def kernel(x, W_temp3, W_toqk, W_tov, W_out, b_out):
    x2 = x[0]
    nsteps = _NBLK + _HEADS + _NBLK
    out = pl.pallas_call(
        _mono_kernel,
        grid=(nsteps,),
        in_specs=[
            pl.BlockSpec((_TBLK, _E), lambda s: (jnp.minimum(s, _NBLK - 1), 0)),
            pl.BlockSpec((_DIM, _E), lambda s: (0, 0)),
            pl.BlockSpec((_DIM_HEADS, _DIM), lambda s: (0, 0)),
            pl.BlockSpec((_DIM_HEADS, _DIM), lambda s: (0, 0)),
            pl.BlockSpec((_OUP, _DIM_HEADS), lambda s: (0, 0)),
            pl.BlockSpec((1, _OUP), lambda s: (0, 0)),
        ],
        out_specs=pl.BlockSpec(
            (_TBLK, _OUP),
            lambda s: (jnp.clip(s - (_NBLK + _HEADS), 0, _NBLK - 1), 0)),
        out_shape=jax.ShapeDtypeStruct((_T, _OUP), jnp.float32),
        scratch_shapes=[
            pltpu.VMEM((_HEADS, _AUG, _T), jnp.bfloat16),
            pltpu.VMEM((_HEADS, _AUG, _T), jnp.bfloat16),
            pltpu.VMEM((_HEADS, _AUG, _T), jnp.bfloat16),
            pltpu.VMEM((_HEADS, _DH, _T), jnp.float32),
        ],
    )(x2, W_temp3, W_toqk, W_tov, W_out, b_out.reshape(1, _OUP))

    return out.reshape(1, _T, _OUP)


# speedup vs baseline: 2.1464x; 1.0062x over previous
import jax
import jax.numpy as jnp
from jax.experimental import pallas as pl
from jax.experimental.pallas import tpu as pltpu

_T = 2048
_E = 768
_DIM = 1024
_HEADS = 16
_DH = 32
_DIM_HEADS = _HEADS * _DH
_OUP = 1024
_TBLK = 512
_NBLK = _T // _TBLK
_LOG2E = 1.4426950408889634
_C = _DH ** -0.5 * _LOG2E
_AUG = _DH + 8


def _mono_kernel(x_ref, wt3_ref, wqk_ref, wv_ref, wo_ref, b_ref, out_ref,
                 lhs_s, rhs_s, va_s, o_s):
    step = pl.program_id(0)

    def proj(i):
        hT = jax.lax.dot_general(
            wt3_ref[...], x_ref[...], (((1,), (1,)), ((), ())),
            preferred_element_type=jnp.float32)
        qkT = jax.lax.dot_general(
            wqk_ref[...], hT, (((1,), (0,)), ((), ())),
            preferred_element_type=jnp.float32)
        vT = jax.lax.dot_general(
            wv_ref[...], hT, (((1,), (0,)), ((), ())),
            preferred_element_type=jnp.float32)
        hid = jax.lax.broadcasted_iota(jnp.int32, (_HEADS, _DIM_HEADS), 0)
        row = jax.lax.broadcasted_iota(jnp.int32, (_HEADS, _DIM_HEADS), 1)
        selT = (row // _DH == hid).astype(jnp.float32)
        nsqT = jax.lax.dot_general(
            selT, qkT * qkT, (((1,), (0,)), ((), ())),
            preferred_element_type=jnp.float32)
        normT = jnp.maximum(jnp.sqrt(nsqT), 1e-12)
        invbT = jax.lax.dot_general(
            selT, 1.0 / normT, (((0,), (0,)), ((), ())),
            preferred_element_type=jnp.float32)
        kT = qkT * invbT

        def split(z):
            return z.reshape(_HEADS, _DH, _TBLK)

        e0 = (jax.lax.broadcasted_iota(jnp.int32, (_HEADS, 8, _TBLK), 1) == 0)
        e0 = e0.astype(jnp.float32)
        mrow = jnp.broadcast_to((-_C * normT)[:, None, :], (_HEADS, 8, _TBLK))
        sl = slice(i * _TBLK, (i + 1) * _TBLK)
        lhs_s[:, :, sl] = jnp.concatenate(
            [split(_C * qkT), mrow * e0], axis=1).astype(jnp.bfloat16)
        rhs_s[:, :, sl] = jnp.concatenate(
            [split(kT), e0], axis=1).astype(jnp.bfloat16)
        va_s[:, :, sl] = jnp.concatenate(
            [split(vT), e0], axis=1).astype(jnp.bfloat16)

    for i in range(_NBLK):
        @pl.when(step == i)
        def _(i=i):
            proj(i)

    @pl.when((step >= _NBLK) & (step < _NBLK + _HEADS))
    def _attn():
        h = step - _NBLK
        lhs = lhs_s[h]
        rhs = rhs_s[h]
        va = va_s[h]
        sT = jax.lax.dot_general(
            rhs, lhs, (((0,), (0,)), ((), ())),
            preferred_element_type=jnp.float32)
        eT = jnp.exp2(sT.astype(jnp.bfloat16))
        numT = jax.lax.dot_general(
            va, eT, (((1,), (0,)), ((), ())),
            preferred_element_type=jnp.float32)
        denom = numT[_DH:_DH + 1, :] - 1.0
        o_s[h] = (numT[:_DH, :] - va[:_DH, :].astype(jnp.float32)) / denom

    for i in range(_NBLK):
        @pl.when(step == _NBLK + _HEADS + i)
        def _(i=i):
            oT = o_s[:, :, i * _TBLK:(i + 1) * _TBLK].reshape(
                _DIM_HEADS, _TBLK)
            out_ref[...] = jax.lax.dot_general(
                oT, wo_ref[...], (((0,), (1,)), ((), ())),
                preferred_element_type=jnp.float32) + b_ref[...]


def kernel(x, W_temp3, W_toqk, W_tov, W_out, b_out):
    x2 = x[0]
    nsteps = _NBLK + _HEADS + _NBLK
    out = pl.pallas_call(
        _mono_kernel,
        grid=(nsteps,),
        in_specs=[
            pl.BlockSpec((_TBLK, _E), lambda s: (jnp.minimum(s, _NBLK - 1), 0)),
            pl.BlockSpec((_DIM, _E), lambda s: (0, 0)),
            pl.BlockSpec((_DIM_HEADS, _DIM), lambda s: (0, 0)),
            pl.BlockSpec((_DIM_HEADS, _DIM), lambda s: (0, 0)),
            pl.BlockSpec((_OUP, _DIM_HEADS), lambda s: (0, 0)),
            pl.BlockSpec((1, _OUP), lambda s: (0, 0)),
        ],
        out_specs=pl.BlockSpec(
            (_TBLK, _OUP),
            lambda s: (jnp.clip(s - (_NBLK + _HEADS), 0, _NBLK - 1), 0)),
        out_shape=jax.ShapeDtypeStruct((_T, _OUP), jnp.float32),
        scratch_shapes=[
            pltpu.VMEM((_HEADS, _AUG, _T), jnp.bfloat16),
            pltpu.VMEM((_HEADS, _AUG, _T), jnp.bfloat16),
            pltpu.VMEM((_HEADS, _AUG, _T), jnp.bfloat16),
            pltpu.VMEM((_HEADS, _DH, _T), jnp.float32),
        ],
    )(x2, W_temp3, W_toqk, W_tov, W_out, b_out.reshape(1, _OUP))

    return out.reshape(1, _T, _OUP)
